# baseline (speedup 1.0000x reference)
import jax
import jax.numpy as jnp
from jax import lax
from jax.experimental import pallas as pl
from jax.experimental.pallas import tpu as pltpu
from jax.experimental.pallas import tpu_sc as plsc

NPOINT = 512
RADIUS = 0.2
NSAMPLE = 32
EPS = 1e-5

_S_TILE = 64
_R_TILE = 4096
_GCHUNK = 128
_NW = 32


def _fps_kernel(xyzt_ref, far0_ref, cent_ref, newx_ref):
    x = xyzt_ref[:, 0, :]
    y = xyzt_ref[:, 1, :]
    z = xyzt_ref[:, 2, :]
    B, N = x.shape
    lane = jax.lax.broadcasted_iota(jnp.int32, (B, N), 1)
    scol = jax.lax.broadcasted_iota(jnp.int32, (B, NPOINT), 1)

    def body(i, state):
        dist_acc, far, cent, nx, ny, nz = state
        sel = lane == far
        cx = jnp.sum(jnp.where(sel, x, 0.0), axis=1, keepdims=True)
        cy = jnp.sum(jnp.where(sel, y, 0.0), axis=1, keepdims=True)
        cz = jnp.sum(jnp.where(sel, z, 0.0), axis=1, keepdims=True)
        hit = scol == i
        cent = jnp.where(hit, far, cent)
        nx = jnp.where(hit, cx, nx)
        ny = jnp.where(hit, cy, ny)
        nz = jnp.where(hit, cz, nz)
        d = (x - cx) ** 2 + (y - cy) ** 2 + (z - cz) ** 2
        dist_acc = jnp.where(d < dist_acc, d, dist_acc)
        m = jnp.max(dist_acc, axis=1, keepdims=True)
        far = jnp.min(jnp.where(dist_acc == m, lane, N), axis=1, keepdims=True)
        return (dist_acc, far, cent, nx, ny, nz)

    init = (
        jnp.full((B, N), 1e10, jnp.float32),
        far0_ref[:, :],
        jnp.zeros((B, NPOINT), jnp.int32),
        jnp.zeros((B, NPOINT), jnp.float32),
        jnp.zeros((B, NPOINT), jnp.float32),
        jnp.zeros((B, NPOINT), jnp.float32),
    )
    _, _, cent, nx, ny, nz = jax.lax.fori_loop(0, NPOINT, body, init)
    cent_ref[:, :] = cent
    newx_ref[:, 0, :] = nx
    newx_ref[:, 1, :] = ny
    newx_ref[:, 2, :] = nz


def _cumsum_lanes(x):
    n = x.shape[-1]
    shift = 1
    while shift < n:
        shifted = jnp.concatenate(
            [jnp.zeros(x.shape[:-1] + (shift,), x.dtype), x[..., : n - shift]],
            axis=-1,
        )
        x = x + shifted
        shift *= 2
    return x


def _group_idx_kernel(xyzt_ref, new_ref, idx_ref):
    b = pl.program_id(0)
    x = xyzt_ref[0, 0:1, :]
    y = xyzt_ref[0, 1:2, :]
    z = xyzt_ref[0, 2:3, :]
    N = x.shape[1]
    new_tile = new_ref[0]
    sx = new_tile[:, 0:1]
    sy = new_tile[:, 1:2]
    sz = new_tile[:, 2:3]
    d = (sx - x) ** 2 + (sy - y) ** 2 + (sz - z) ** 2
    mask = d <= RADIUS ** 2
    c = _cumsum_lanes(mask.astype(jnp.int32))
    count = c[:, N - 1 : N]

    kk = jax.lax.broadcasted_iota(jnp.int32, (_S_TILE, NSAMPLE), 1)
    lte = (c[:, None, :] <= kk[:, :, None]).astype(jnp.float32)
    idx = jnp.sum(lte, axis=2).astype(jnp.int32)
    idx = jnp.where(kk < count, idx, idx[:, 0:1])
    idx_ref[...] = idx + b * N


def _sc_gather_kernel(table_hbm, idx_hbm, out_hbm, idx_v, rows_v, sem):
    wid = lax.axis_index("s") * 2 + lax.axis_index("c")
    rows_per_w = out_hbm.shape[0] // _NW
    base = wid * rows_per_w

    def body(j, carry):
        off = base + j * _GCHUNK
        pltpu.sync_copy(idx_hbm.at[pl.ds(off, _GCHUNK)], idx_v)
        pltpu.async_copy(table_hbm.at[idx_v], rows_v, sem).wait()
        pltpu.sync_copy(rows_v, out_hbm.at[pl.ds(off, _GCHUNK)])
        return carry

    jax.lax.fori_loop(0, rows_per_w // _GCHUNK, body, 0)


def _prep_kernel(graw_ref, cen_ref, feat_ref, stats_ref):
    t = pl.program_id(0)
    graw = graw_ref[...]
    C = graw.shape[1]
    ng = _R_TILE // NSAMPLE
    cen = cen_ref[...]
    cenp = jnp.concatenate([cen, jnp.zeros((ng, C - 3), jnp.float32)], axis=1)
    g3 = graw.reshape(ng, NSAMPLE, C) - cenp[:, None, :]
    f = g3.reshape(_R_TILE, C)
    feat_ref[...] = f

    @pl.when(t == 0)
    def _():
        stats_ref[...] = jnp.zeros_like(stats_ref)

    s_sum = jnp.sum(f, axis=0, keepdims=True)
    gram = jax.lax.dot_general(
        f, f, (((0,), (0,)), ((), ())),
        preferred_element_type=jnp.float32,
    )
    stats_ref[0:1, :] += s_sum
    stats_ref[1:, :] += gram


def _mlp_kernel(z_ref, w_ref, b_ref, out_ref, stats_ref):
    t = pl.program_id(0)
    z = z_ref[...]
    o = jax.lax.dot_general(
        z, w_ref[...], (((1,), (1,)), ((), ())),
        preferred_element_type=jnp.float32,
    ) + b_ref[...]
    o = jnp.maximum(o, 0.0)
    out_ref[...] = o

    @pl.when(t == 0)
    def _():
        stats_ref[...] = jnp.zeros_like(stats_ref)

    s_sum = jnp.sum(o, axis=0, keepdims=True)
    gram = jax.lax.dot_general(
        o, o, (((0,), (0,)), ((), ())),
        preferred_element_type=jnp.float32,
    )
    stats_ref[0:1, :] += s_sum
    stats_ref[1:, :] += gram


def _mlp_pool_kernel(z_ref, w_ref, b_ref, out_ref):
    z = z_ref[...]
    o = jax.lax.dot_general(
        z, w_ref[...], (((1,), (1,)), ((), ())),
        preferred_element_type=jnp.float32,
    ) + b_ref[...]
    o = jnp.maximum(o, 0.0)
    R, C = o.shape
    o = o.reshape(R // NSAMPLE, NSAMPLE, C)
    out_ref[...] = jnp.max(o, axis=1)


def _bn_affine(W, bias, g, beta, s_sum, gram, P):
    ws = W @ s_sum
    mean = (ws + P * bias) / P
    q = jnp.sum((W @ gram) * W, axis=1)
    ex2 = (q + 2.0 * bias * ws + P * bias * bias) / P
    var = ex2 - mean * mean
    scale = g / jnp.sqrt(var + EPS)
    return W * scale[:, None], (bias - mean) * scale + beta


def kernel(xyz, points, conv_w0, conv_b0, bn_g0, bn_b0, conv_w1, conv_b1,
           bn_g1, bn_b1, conv_w2, conv_b2, bn_g2, bn_b2):
    B, N, _ = xyz.shape
    C = points.shape[2]
    Cin = 3 + C
    S = NPOINT
    P = B * S * NSAMPLE

    xyzt = jnp.transpose(xyz, (0, 2, 1))
    far0 = jax.random.randint(jax.random.key(1), (B, 1), 0, N, dtype=jnp.int32)

    cent, newx = pl.pallas_call(
        _fps_kernel,
        out_shape=(
            jax.ShapeDtypeStruct((B, S), jnp.int32),
            jax.ShapeDtypeStruct((B, 3, S), jnp.float32),
        ),
    )(xyzt, far0)
    new_xyz = jnp.transpose(newx, (0, 2, 1))

    gidx = pl.pallas_call(
        _group_idx_kernel,
        grid=(B, S // _S_TILE),
        in_specs=[
            pl.BlockSpec((1, 3, N), lambda b, st: (b, 0, 0)),
            pl.BlockSpec((1, _S_TILE, 3), lambda b, st: (b, st, 0)),
        ],
        out_specs=pl.BlockSpec(
            (_S_TILE, NSAMPLE), lambda b, st: (b * (S // _S_TILE) + st, 0)
        ),
        out_shape=jax.ShapeDtypeStruct((B * S, NSAMPLE), jnp.int32),
    )(xyzt, new_xyz)

    table = jnp.concatenate([xyz, points], axis=2).reshape(B * N, Cin)
    gidx_flat = gidx.reshape(P)

    mesh = plsc.VectorSubcoreMesh(core_axis_name="c", subcore_axis_name="s")
    g_raw = pl.kernel(
        _sc_gather_kernel,
        mesh=mesh,
        compiler_params=pltpu.CompilerParams(use_tc_tiling_on_sc=False),
        out_type=jax.ShapeDtypeStruct((P, Cin), jnp.float32),
        scratch_types=[
            pltpu.VMEM((_GCHUNK,), jnp.int32),
            pltpu.VMEM((_GCHUNK, Cin), jnp.float32),
            pltpu.SemaphoreType.DMA,
        ],
    )(table, gidx_flat)

    cen_all = new_xyz.reshape(B * S, 3)
    z, stats0 = pl.pallas_call(
        _prep_kernel,
        grid=(P // _R_TILE,),
        in_specs=[
            pl.BlockSpec((_R_TILE, Cin), lambda t: (t, 0)),
            pl.BlockSpec((_R_TILE // NSAMPLE, 3), lambda t: (t, 0)),
        ],
        out_specs=(
            pl.BlockSpec((_R_TILE, Cin), lambda t: (t, 0)),
            pl.BlockSpec((1 + Cin, Cin), lambda t: (0, 0)),
        ),
        out_shape=(
            jax.ShapeDtypeStruct((P, Cin), jnp.float32),
            jax.ShapeDtypeStruct((1 + Cin, Cin), jnp.float32),
        ),
    )(g_raw, cen_all)

    params = [
        (conv_w0, conv_b0, bn_g0, bn_b0),
        (conv_w1, conv_b1, bn_g1, bn_b1),
        (conv_w2, conv_b2, bn_g2, bn_b2),
    ]

    s_sum, gram = stats0[0], stats0[1:]
    for li in range(2):
        W, bias, gg, beta = params[li]
        W2, b2 = _bn_affine(W, bias, gg, beta, s_sum, gram, P)
        Cout = W2.shape[0]
        z, stats = pl.pallas_call(
            _mlp_kernel,
            grid=(P // _R_TILE,),
            in_specs=[
                pl.BlockSpec((_R_TILE, z.shape[1]), lambda t: (t, 0)),
                pl.BlockSpec((Cout, z.shape[1]), lambda t: (0, 0)),
                pl.BlockSpec((1, Cout), lambda t: (0, 0)),
            ],
            out_specs=(
                pl.BlockSpec((_R_TILE, Cout), lambda t: (t, 0)),
                pl.BlockSpec((1 + Cout, Cout), lambda t: (0, 0)),
            ),
            out_shape=(
                jax.ShapeDtypeStruct((P, Cout), jnp.float32),
                jax.ShapeDtypeStruct((1 + Cout, Cout), jnp.float32),
            ),
        )(z, W2, b2[None, :])
        s_sum, gram = stats[0], stats[1:]

    W, bias, gg, beta = params[2]
    W2, b2 = _bn_affine(W, bias, gg, beta, s_sum, gram, P)
    Cout = W2.shape[0]
    pooled = pl.pallas_call(
        _mlp_pool_kernel,
        grid=(P // _R_TILE,),
        in_specs=[
            pl.BlockSpec((_R_TILE, z.shape[1]), lambda t: (t, 0)),
            pl.BlockSpec((Cout, z.shape[1]), lambda t: (0, 0)),
            pl.BlockSpec((1, Cout), lambda t: (0, 0)),
        ],
        out_specs=pl.BlockSpec((_R_TILE // NSAMPLE, Cout), lambda t: (t, 0)),
        out_shape=jax.ShapeDtypeStruct((B * S, Cout), jnp.float32),
    )(z, W2, b2[None, :])

    new_points = pooled.reshape(B, S, Cout)
    return (new_xyz, new_points)


# speedup vs baseline: 1.0610x; 1.0610x over previous
import jax
import jax.numpy as jnp
from jax.experimental import pallas as pl
from jax.experimental.pallas import tpu as pltpu

NPOINT = 512
RADIUS = 0.2
NSAMPLE = 32
EPS = 1e-5

_S_TILE = 64
_R_TILE = 4096


def _fps_kernel(xyzt_ref, far0_ref, cent_ref, newx_ref):
    x = xyzt_ref[:, 0, :]
    y = xyzt_ref[:, 1, :]
    z = xyzt_ref[:, 2, :]
    B, N = x.shape
    lane = jax.lax.broadcasted_iota(jnp.int32, (B, N), 1)
    scol = jax.lax.broadcasted_iota(jnp.int32, (B, NPOINT), 1)

    def body(i, state):
        dist_acc, far, cent, nx, ny, nz = state
        sel = lane == far
        cx = jnp.sum(jnp.where(sel, x, 0.0), axis=1, keepdims=True)
        cy = jnp.sum(jnp.where(sel, y, 0.0), axis=1, keepdims=True)
        cz = jnp.sum(jnp.where(sel, z, 0.0), axis=1, keepdims=True)
        hit = scol == i
        cent = jnp.where(hit, far, cent)
        nx = jnp.where(hit, cx, nx)
        ny = jnp.where(hit, cy, ny)
        nz = jnp.where(hit, cz, nz)
        d = (x - cx) ** 2 + (y - cy) ** 2 + (z - cz) ** 2
        dist_acc = jnp.where(d < dist_acc, d, dist_acc)
        m = jnp.max(dist_acc, axis=1, keepdims=True)
        far = jnp.min(jnp.where(dist_acc == m, lane, N), axis=1, keepdims=True)
        return (dist_acc, far, cent, nx, ny, nz)

    init = (
        jnp.full((B, N), 1e10, jnp.float32),
        far0_ref[:, :],
        jnp.zeros((B, NPOINT), jnp.int32),
        jnp.zeros((B, NPOINT), jnp.float32),
        jnp.zeros((B, NPOINT), jnp.float32),
        jnp.zeros((B, NPOINT), jnp.float32),
    )
    _, _, cent, nx, ny, nz = jax.lax.fori_loop(0, NPOINT, body, init)
    cent_ref[:, :] = cent
    newx_ref[:, 0, :] = nx
    newx_ref[:, 1, :] = ny
    newx_ref[:, 2, :] = nz


def _cumsum_lanes(x):
    n = x.shape[-1]
    shift = 1
    while shift < n:
        shifted = jnp.concatenate(
            [jnp.zeros(x.shape[:-1] + (shift,), x.dtype), x[..., : n - shift]],
            axis=-1,
        )
        x = x + shifted
        shift *= 2
    return x


def _group_kernel(xyzt_ref, feat_ref, new_ref, g_ref, stats_ref):
    b = pl.program_id(0)
    st = pl.program_id(1)
    x = xyzt_ref[0, 0:1, :]
    y = xyzt_ref[0, 1:2, :]
    z = xyzt_ref[0, 2:3, :]
    N = x.shape[1]
    H = N // 2
    new_tile = new_ref[0]
    sx = new_tile[:, 0:1]
    sy = new_tile[:, 1:2]
    sz = new_tile[:, 2:3]
    kk = jax.lax.broadcasted_iota(jnp.int32, (_S_TILE, NSAMPLE, 1), 1)
    kkf = (kk + 1).astype(jnp.float32)
    feats = feat_ref[0]

    def rank_range(lo, width):
        xh = x[:, lo : lo + width]
        yh = y[:, lo : lo + width]
        zh = z[:, lo : lo + width]
        d = (sx - xh) ** 2 + (sy - yh) ** 2 + (sz - zh) ** 2
        mask = d <= RADIUS ** 2
        c = _cumsum_lanes(mask.astype(jnp.int32))
        return mask, c

    def onehot_mm(cm_f32, feats_part):
        onehot = jnp.where(cm_f32[:, None, :] == kkf, 1.0, 0.0)
        onehot = onehot.reshape(_S_TILE * NSAMPLE, cm_f32.shape[1])
        return jnp.dot(onehot, feats_part, preferred_element_type=jnp.float32)

    mask1, c1 = rank_range(0, H)
    cnt1 = c1[:, H - 1 : H]

    def fast_fn():
        cm1 = jnp.where(mask1, c1, 0).astype(jnp.float32)
        return onehot_mm(cm1, feats[:H]), jnp.minimum(cnt1, NSAMPLE)

    def slow_fn():
        mask2, c2 = rank_range(H, H)
        c2 = c2 + cnt1
        cm = jnp.concatenate(
            [jnp.where(mask1, c1, 0), jnp.where(mask2, c2, 0)], axis=1
        ).astype(jnp.float32)
        return onehot_mm(cm, feats), c2[:, H - 1 : H]

    g, count = jax.lax.cond(jnp.min(cnt1) >= NSAMPLE, fast_fn, slow_fn)

    C = g.shape[1]
    g3 = g.reshape(_S_TILE, NSAMPLE, C)
    first = g3[:, 0:1, :]
    valid = kk < count[:, :, None]
    g3 = jnp.where(valid, g3, first)
    cen = jnp.concatenate(
        [new_tile, jnp.zeros((_S_TILE, C - 3), jnp.float32)], axis=1
    )
    g3 = g3 - cen[:, None, :]
    g_flat = g3.reshape(_S_TILE * NSAMPLE, C)
    g_ref[0] = g_flat

    @pl.when((b == 0) & (st == 0))
    def _():
        stats_ref[...] = jnp.zeros_like(stats_ref)

    s_sum = jnp.sum(g_flat, axis=0, keepdims=True)
    gram = jax.lax.dot_general(
        g_flat, g_flat, (((0,), (0,)), ((), ())),
        preferred_element_type=jnp.float32,
    )
    stats_ref[0:1, :] += s_sum
    stats_ref[1:, :] += gram


def _mlp_kernel(z_ref, w_ref, b_ref, out_ref, stats_ref):
    t = pl.program_id(0)
    z = z_ref[...]
    o = jax.lax.dot_general(
        z, w_ref[...], (((1,), (1,)), ((), ())),
        preferred_element_type=jnp.float32,
    ) + b_ref[...]
    o = jnp.maximum(o, 0.0)
    out_ref[...] = o

    @pl.when(t == 0)
    def _():
        stats_ref[...] = jnp.zeros_like(stats_ref)

    s_sum = jnp.sum(o, axis=0, keepdims=True)
    gram = jax.lax.dot_general(
        o, o, (((0,), (0,)), ((), ())),
        preferred_element_type=jnp.float32,
    )
    stats_ref[0:1, :] += s_sum
    stats_ref[1:, :] += gram


def _mlp_pool_kernel(z_ref, w_ref, b_ref, out_ref):
    z = z_ref[...]
    o = jax.lax.dot_general(
        z, w_ref[...], (((1,), (1,)), ((), ())),
        preferred_element_type=jnp.float32,
    ) + b_ref[...]
    o = jnp.maximum(o, 0.0)
    R, C = o.shape
    o = o.reshape(R // NSAMPLE, NSAMPLE, C)
    out_ref[...] = jnp.max(o, axis=1)


def _bn_affine(W, bias, g, beta, s_sum, gram, P):
    ws = W @ s_sum
    mean = (ws + P * bias) / P
    q = jnp.sum((W @ gram) * W, axis=1)
    ex2 = (q + 2.0 * bias * ws + P * bias * bias) / P
    var = ex2 - mean * mean
    scale = g / jnp.sqrt(var + EPS)
    return W * scale[:, None], (bias - mean) * scale + beta


def kernel(xyz, points, conv_w0, conv_b0, bn_g0, bn_b0, conv_w1, conv_b1,
           bn_g1, bn_b1, conv_w2, conv_b2, bn_g2, bn_b2):
    B, N, _ = xyz.shape
    C = points.shape[2]
    Cin = 3 + C
    S = NPOINT
    P = B * S * NSAMPLE

    xyzt = jnp.transpose(xyz, (0, 2, 1))
    far0 = jax.random.randint(jax.random.key(1), (B, 1), 0, N, dtype=jnp.int32)

    cent, newx = pl.pallas_call(
        _fps_kernel,
        out_shape=(
            jax.ShapeDtypeStruct((B, S), jnp.int32),
            jax.ShapeDtypeStruct((B, 3, S), jnp.float32),
        ),
    )(xyzt, far0)
    new_xyz = jnp.transpose(newx, (0, 2, 1))

    featcat = jnp.concatenate([xyz, points], axis=2)

    g, stats0 = pl.pallas_call(
        _group_kernel,
        grid=(B, S // _S_TILE),
        in_specs=[
            pl.BlockSpec((1, 3, N), lambda b, st: (b, 0, 0)),
            pl.BlockSpec((1, N, Cin), lambda b, st: (b, 0, 0)),
            pl.BlockSpec((1, _S_TILE, 3), lambda b, st: (b, st, 0)),
        ],
        out_specs=(
            pl.BlockSpec((1, _S_TILE * NSAMPLE, Cin), lambda b, st: (b, st, 0)),
            pl.BlockSpec((1 + Cin, Cin), lambda b, st: (0, 0)),
        ),
        out_shape=(
            jax.ShapeDtypeStruct((B, S * NSAMPLE, Cin), jnp.float32),
            jax.ShapeDtypeStruct((1 + Cin, Cin), jnp.float32),
        ),
    )(xyzt, featcat, new_xyz)

    z = g.reshape(B * S * NSAMPLE, Cin)

    params = [
        (conv_w0, conv_b0, bn_g0, bn_b0),
        (conv_w1, conv_b1, bn_g1, bn_b1),
        (conv_w2, conv_b2, bn_g2, bn_b2),
    ]

    s_sum, gram = stats0[0], stats0[1:]
    for li in range(2):
        W, bias, gg, beta = params[li]
        W2, b2 = _bn_affine(W, bias, gg, beta, s_sum, gram, P)
        Cout = W2.shape[0]
        z, stats = pl.pallas_call(
            _mlp_kernel,
            grid=(P // _R_TILE,),
            in_specs=[
                pl.BlockSpec((_R_TILE, z.shape[1]), lambda t: (t, 0)),
                pl.BlockSpec((Cout, z.shape[1]), lambda t: (0, 0)),
                pl.BlockSpec((1, Cout), lambda t: (0, 0)),
            ],
            out_specs=(
                pl.BlockSpec((_R_TILE, Cout), lambda t: (t, 0)),
                pl.BlockSpec((1 + Cout, Cout), lambda t: (0, 0)),
            ),
            out_shape=(
                jax.ShapeDtypeStruct((P, Cout), jnp.float32),
                jax.ShapeDtypeStruct((1 + Cout, Cout), jnp.float32),
            ),
        )(z, W2, b2[None, :])
        s_sum, gram = stats[0], stats[1:]

    W, bias, gg, beta = params[2]
    W2, b2 = _bn_affine(W, bias, gg, beta, s_sum, gram, P)
    Cout = W2.shape[0]
    pooled = pl.pallas_call(
        _mlp_pool_kernel,
        grid=(P // _R_TILE,),
        in_specs=[
            pl.BlockSpec((_R_TILE, z.shape[1]), lambda t: (t, 0)),
            pl.BlockSpec((Cout, z.shape[1]), lambda t: (0, 0)),
            pl.BlockSpec((1, Cout), lambda t: (0, 0)),
        ],
        out_specs=pl.BlockSpec((_R_TILE // NSAMPLE, Cout), lambda t: (t, 0)),
        out_shape=jax.ShapeDtypeStruct((B * S, Cout), jnp.float32),
    )(z, W2, b2[None, :])

    new_points = pooled.reshape(B, S, Cout)
    return (new_xyz, new_points)
